# baseline (speedup 1.0000x reference)
import math

import jax
import jax.numpy as jnp
from jax.experimental import pallas as pl


def _cmul(ar, ai, br, bi):
    return ar * br - ai * bi, ar * bi + ai * br


def _build_powers(dr, di, e0r, e0i, nrows):
    er, ei = e0r, e0i
    sr, si = dr, di
    rows = 1
    while rows < nrows:
        take = min(rows, nrows - rows)
        nr, ni = _cmul(er[:take], ei[:take], sr, si)
        er = jnp.concatenate([er, nr], axis=0)
        ei = jnp.concatenate([ei, ni], axis=0)
        if 2 * rows < nrows:
            sr, si = _cmul(sr, si, sr, si)
        rows += take
    return er, ei


def _nufft_block_kernel(fpur_ref, fpui_ref, fpvr_ref, fpvi_ref,
                        fmur_ref, fmui_ref, fmvr_ref, fmvi_ref,
                        kx_ref, ky_ref, sdcf_ref, yr_ref, yi_ref):
    G = fpur_ref.shape[0]
    tw = -2.0 * math.pi
    ax = tw * kx_ref[0]
    ay = tw * ky_ref[0]
    a2 = jnp.concatenate([ax, ay], axis=0)
    c2 = jnp.cos(a2)
    s2 = jnp.sin(a2)
    dxr, dyr = c2[0:1], c2[1:2]
    dxi, dyi = s2[0:1], s2[1:2]

    one = jnp.ones_like(ax)
    zero = jnp.zeros_like(ax)
    w = sdcf_ref[0]
    cx, sx = _build_powers(dxr, dxi, one, zero, G)
    cy, sy = _build_powers(dyr, dyi, w, zero, G)

    def dot(a_ref, b):
        return jnp.dot(a_ref[...], b, preferred_element_type=jnp.float32)

    ur = dot(fpur_ref, cx) - dot(fpvi_ref, sx)
    ui = dot(fpui_ref, cx) + dot(fpvr_ref, sx)
    vr = dot(fmur_ref, cx) - dot(fmvi_ref, sx)
    vi = dot(fmui_ref, cx) + dot(fmvr_ref, sx)
    yr_ref[0, 0, :] = jnp.sum(ur * cy - vi * sy, axis=0)
    yi_ref[0, 0, :] = jnp.sum(ui * cy + vr * sy, axis=0)


def _fold_weights(x):
    N = x.shape[0]
    G = N // 2 + 8
    xrt = x[..., 0].T
    xit = x[..., 1].T

    def cfold(m):
        a = m[:, N // 2:]
        b = m[:, N // 2:0:-1]
        zp = jnp.zeros((m.shape[0], G - N // 2 - 1), jnp.float32)
        plus = jnp.concatenate([a + b, m[:, 0:1], zp], axis=1)
        minus = jnp.concatenate([a - b, -m[:, 0:1], zp], axis=1)
        return plus, minus

    def rfold(m):
        a = m[N // 2:, :]
        b = m[N // 2:0:-1, :]
        zp = jnp.zeros((G - N // 2 - 1, G), jnp.float32)
        plus = jnp.concatenate([a + b, m[0:1, :], zp], axis=0)
        minus = jnp.concatenate([a - b, -m[0:1, :], zp], axis=0)
        return plus, minus

    ur, vr = cfold(xrt)
    ui, vi = cfold(xit)
    fpur, fmur = rfold(ur)
    fpui, fmui = rfold(ui)
    fpvr, fmvr = rfold(vr)
    fpvi, fmvi = rfold(vi)
    half_col = jnp.ones((1, G), jnp.float32).at[0, 0].set(0.5)
    half_row = half_col.T
    fp = [f * half_col * half_row for f in (fpur, fpui, fpvr, fpvi)]
    fm = [f * half_col for f in (fmur, fmui, fmvr, fmvi)]
    return fp + fm, G


def kernel(x, trajectory, dcf):
    K = trajectory.shape[1]
    S = 8192 if K % 8192 == 0 else K
    nblk = K // S
    fmats, G = _fold_weights(x)
    kx = trajectory[0].reshape(nblk, 1, S)
    ky = trajectory[1].reshape(nblk, 1, S)
    sdcf = jnp.sqrt(dcf).reshape(nblk, 1, S)
    fspec = pl.BlockSpec((G, G), lambda b: (0, 0))
    rspec = pl.BlockSpec((1, 1, S), lambda b: (b, 0, 0))
    yr, yi = pl.pallas_call(
        _nufft_block_kernel,
        grid=(nblk,),
        in_specs=[fspec] * 8 + [rspec] * 3,
        out_specs=[rspec, rspec],
        out_shape=[
            jax.ShapeDtypeStruct((nblk, 1, S), jnp.float32),
            jax.ShapeDtypeStruct((nblk, 1, S), jnp.float32),
        ],
    )(*fmats, kx, ky, sdcf)
    return jnp.stack([yr.reshape(K), yi.reshape(K)], axis=-1)


# speedup vs baseline: 1.0619x; 1.0619x over previous
import math

import jax
import jax.numpy as jnp
from jax.experimental import pallas as pl


def _cmul(ar, ai, br, bi):
    return ar * br - ai * bi, ar * bi + ai * br


def _build_powers(dr, di, e0r, e0i, nrows):
    er, ei = e0r, e0i
    sr, si = dr, di
    rows = 1
    while rows < nrows:
        take = min(rows, nrows - rows)
        nr, ni = _cmul(er[:take], ei[:take], sr, si)
        er = jnp.concatenate([er, nr], axis=0)
        ei = jnp.concatenate([ei, ni], axis=0)
        if 2 * rows < nrows:
            sr, si = _cmul(sr, si, sr, si)
        rows += take
    return er, ei


def _nufft_block_kernel(fpur_ref, fpui_ref, fpvr_ref, fpvi_ref,
                        fmur_ref, fmui_ref, fmvr_ref, fmvi_ref,
                        kx_ref, ky_ref, sdcf_ref, yr_ref, yi_ref):
    G = fpur_ref.shape[0]
    tw = -2.0 * math.pi
    ax = tw * kx_ref[0]
    ay = tw * ky_ref[0]
    a2 = jnp.concatenate([ax, ay], axis=0)
    c2 = jnp.cos(a2)
    s2 = jnp.sin(a2)
    dxr, dyr = c2[0:1], c2[1:2]
    dxi, dyi = s2[0:1], s2[1:2]

    one = jnp.ones_like(ax)
    zero = jnp.zeros_like(ax)
    w = sdcf_ref[0]
    cx, sx = _build_powers(dxr, dxi, one, zero, G)
    cy, sy = _build_powers(dyr, dyi, w, zero, G)

    def dot(a_ref, b):
        return jnp.dot(a_ref[...].astype(jnp.bfloat16), b.astype(jnp.bfloat16),
                       preferred_element_type=jnp.float32)

    ur = dot(fpur_ref, cx) - dot(fpvi_ref, sx)
    ui = dot(fpui_ref, cx) + dot(fpvr_ref, sx)
    vr = dot(fmur_ref, cx) - dot(fmvi_ref, sx)
    vi = dot(fmui_ref, cx) + dot(fmvr_ref, sx)
    yr_ref[0, 0, :] = jnp.sum(ur * cy - vi * sy, axis=0)
    yi_ref[0, 0, :] = jnp.sum(ui * cy + vr * sy, axis=0)


def _fold_weights(x):
    N = x.shape[0]
    G = N // 2 + 8
    xrt = x[..., 0].T
    xit = x[..., 1].T

    def cfold(m):
        a = m[:, N // 2:]
        b = m[:, N // 2:0:-1]
        zp = jnp.zeros((m.shape[0], G - N // 2 - 1), jnp.float32)
        plus = jnp.concatenate([a + b, m[:, 0:1], zp], axis=1)
        minus = jnp.concatenate([a - b, -m[:, 0:1], zp], axis=1)
        return plus, minus

    def rfold(m):
        a = m[N // 2:, :]
        b = m[N // 2:0:-1, :]
        zp = jnp.zeros((G - N // 2 - 1, G), jnp.float32)
        plus = jnp.concatenate([a + b, m[0:1, :], zp], axis=0)
        minus = jnp.concatenate([a - b, -m[0:1, :], zp], axis=0)
        return plus, minus

    ur, vr = cfold(xrt)
    ui, vi = cfold(xit)
    fpur, fmur = rfold(ur)
    fpui, fmui = rfold(ui)
    fpvr, fmvr = rfold(vr)
    fpvi, fmvi = rfold(vi)
    half_col = jnp.ones((1, G), jnp.float32).at[0, 0].set(0.5)
    half_row = half_col.T
    fp = [f * half_col * half_row for f in (fpur, fpui, fpvr, fpvi)]
    fm = [f * half_col for f in (fmur, fmui, fmvr, fmvi)]
    return fp + fm, G


def kernel(x, trajectory, dcf):
    K = trajectory.shape[1]
    S = 4096 if K % 4096 == 0 else K
    nblk = K // S
    fmats, G = _fold_weights(x)
    kx = trajectory[0].reshape(nblk, 1, S)
    ky = trajectory[1].reshape(nblk, 1, S)
    sdcf = jnp.sqrt(dcf).reshape(nblk, 1, S)
    fspec = pl.BlockSpec((G, G), lambda b: (0, 0))
    rspec = pl.BlockSpec((1, 1, S), lambda b: (b, 0, 0))
    yr, yi = pl.pallas_call(
        _nufft_block_kernel,
        grid=(nblk,),
        in_specs=[fspec] * 8 + [rspec] * 3,
        out_specs=[rspec, rspec],
        out_shape=[
            jax.ShapeDtypeStruct((nblk, 1, S), jnp.float32),
            jax.ShapeDtypeStruct((nblk, 1, S), jnp.float32),
        ],
    )(*fmats, kx, ky, sdcf)
    return jnp.stack([yr.reshape(K), yi.reshape(K)], axis=-1)
